# baseline (speedup 1.0000x reference)
import jax
import jax.numpy as jnp
from jax.experimental import pallas as pl

_S = 3
_SH = 3
_NODES = _S + _SH
_NBRS = (
    (1, 2, 3, 0),
    (0, 2, 4, 1),
    (1, 0, 5, 2),
    (0, 3),
    (1, 4),
    (2, 5),
)


_OUT = (
    (0, 1, 2, 3),
    (0, 1, 2, 4),
    (0, 1, 2, 5),
    (0, 3),
    (1, 4),
    (2, 5),
)


def _gat_fused_kernel(spec_ref, shr_ref, w_ref, wa_ref, e_ref, bias_ref,
                      xout_ref, fused_ref):
    bB = spec_ref.shape[1]
    D = spec_ref.shape[2]
    H = wa_ref.shape[1] // 2
    w = w_ref[...]
    wa = wa_ref[...]

    def x_node(n):
        if n < _S:
            return spec_ref[n].astype(jnp.bfloat16)
        return shr_ref[n - _S].astype(jnp.bfloat16)

    a_src = []
    a_dst = []
    for n in range(_NODES):
        a = jnp.dot(x_node(n), wa, preferred_element_type=jnp.float32)
        a_src.append(a[:, 0:H])
        a_dst.append(a[:, H:2 * H])

    coef = {}
    for i in range(_NODES):
        srcs = _NBRS[i]
        ad = a_dst[i]
        al = []
        for j in srcs:
            v = a_src[j] + ad
            al.append(jnp.where(v >= 0, v, 0.2 * v))
        m = al[0]
        for a in al[1:]:
            m = jnp.maximum(m, a)
        exs = [jnp.exp(a - m) for a in al]
        den = exs[0]
        for e in exs[1:]:
            den = den + e
        inv = 1.0 / (den + 1e-16)
        for j, ex in zip(srcs, exs):
            coef[(i, j)] = ex * inv

    acc = [None] * _NODES
    for j in range(_NODES):
        h_j = jnp.dot(x_node(j), w, preferred_element_type=jnp.float32)
        for i in _OUT[j]:
            cb = jnp.dot(coef[(i, j)].astype(jnp.bfloat16), e_ref[...],
                         preferred_element_type=jnp.float32)
            c = cb * h_j
            s = c[:, 0:D]
            for head in range(1, H):
                s = s + c[:, head * D:(head + 1) * D]
            acc[i] = s if acc[i] is None else acc[i] + s

    inv_h = 1.0 / H
    acc_mean = None
    for i in range(_NODES):
        out_i = acc[i] * inv_h + bias_ref[...]
        out_i = jnp.where(out_i > 0, out_i, jnp.exp(jnp.minimum(out_i, 0.0)) - 1.0)
        xout_ref[:, i, :] = out_i
        acc_mean = out_i if acc_mean is None else acc_mean + out_i
    fused_ref[...] = acc_mean * (1.0 / _NODES)


def kernel(specific_features, shared_features, W, att_src, att_dst, bias):
    S, B, D = specific_features.shape
    H = att_src.shape[0]
    bB = 512
    grid = (B // bB,)
    bias2 = bias.reshape(1, D)
    w3 = W.reshape(D, H, D)
    wa = jnp.concatenate([
        jnp.einsum('dhe,he->dh', w3, att_src),
        jnp.einsum('dhe,he->dh', w3, att_dst),
    ], axis=1)
    e_mat = jnp.repeat(jnp.eye(H, dtype=jnp.bfloat16), D, axis=1)

    x_out, fused = pl.pallas_call(
        _gat_fused_kernel,
        grid=grid,
        in_specs=[
            pl.BlockSpec((S, bB, D), lambda i: (0, i, 0)),
            pl.BlockSpec((_SH, bB, D), lambda i: (0, i, 0)),
            pl.BlockSpec((D, H * D), lambda i: (0, 0)),
            pl.BlockSpec((D, 2 * H), lambda i: (0, 0)),
            pl.BlockSpec((H, H * D), lambda i: (0, 0)),
            pl.BlockSpec((1, D), lambda i: (0, 0)),
        ],
        out_specs=[
            pl.BlockSpec((bB, _NODES, D), lambda i: (i, 0, 0)),
            pl.BlockSpec((bB, D), lambda i: (i, 0)),
        ],
        out_shape=[
            jax.ShapeDtypeStruct((B, _NODES, D), jnp.float32),
            jax.ShapeDtypeStruct((B, D), jnp.float32),
        ],
    )(specific_features, shared_features, W.astype(jnp.bfloat16),
      wa.astype(jnp.bfloat16), e_mat, bias2)
    return fused, x_out


# speedup vs baseline: 1.0380x; 1.0380x over previous
import jax
import jax.numpy as jnp
from jax.experimental import pallas as pl

_S = 3
_SH = 3
_NODES = _S + _SH
_NBRS = (
    (1, 2, 3, 0),
    (0, 2, 4, 1),
    (1, 0, 5, 2),
    (0, 3),
    (1, 4),
    (2, 5),
)


_OUT = (
    (0, 1, 2, 3),
    (0, 1, 2, 4),
    (0, 1, 2, 5),
    (0, 3),
    (1, 4),
    (2, 5),
)


def _gat_fused_kernel(spec_ref, shr_ref, w_ref, wa_ref, e_ref, bias_ref,
                      xout_ref, fused_ref):
    bB = spec_ref.shape[1]
    D = spec_ref.shape[2]
    H = wa_ref.shape[1] // 2
    w = w_ref[...]
    wa = wa_ref[...]

    def x_node(n):
        if n < _S:
            return spec_ref[n]
        return shr_ref[n - _S]

    a_src = []
    a_dst = []
    for n in range(_NODES):
        a = jnp.dot(x_node(n), wa, preferred_element_type=jnp.float32)
        a_src.append(a[:, 0:H])
        a_dst.append(a[:, H:2 * H])

    coef = {}
    for i in range(_NODES):
        srcs = _NBRS[i]
        ad = a_dst[i]
        al = []
        for j in srcs:
            v = a_src[j] + ad
            al.append(jnp.where(v >= 0, v, 0.2 * v))
        m = al[0]
        for a in al[1:]:
            m = jnp.maximum(m, a)
        exs = [jnp.exp(a - m) for a in al]
        den = exs[0]
        for e in exs[1:]:
            den = den + e
        inv = 1.0 / (den + 1e-16)
        for j, ex in zip(srcs, exs):
            coef[(i, j)] = ex * inv

    acc = [None] * _NODES
    for j in range(_NODES):
        h_j = jnp.dot(x_node(j), w, preferred_element_type=jnp.float32)
        for i in _OUT[j]:
            cb = jnp.dot(coef[(i, j)], e_ref[...],
                         preferred_element_type=jnp.float32)
            c = cb * h_j
            s = c[:, 0:D]
            for head in range(1, H):
                s = s + c[:, head * D:(head + 1) * D]
            acc[i] = s if acc[i] is None else acc[i] + s

    inv_h = 1.0 / H
    acc_mean = None
    for i in range(_NODES):
        out_i = acc[i] * inv_h + bias_ref[...]
        out_i = jnp.where(out_i > 0, out_i, jnp.exp(jnp.minimum(out_i, 0.0)) - 1.0)
        xout_ref[:, i, :] = out_i
        acc_mean = out_i if acc_mean is None else acc_mean + out_i
    fused_ref[...] = acc_mean * (1.0 / _NODES)


def kernel(specific_features, shared_features, W, att_src, att_dst, bias):
    S, B, D = specific_features.shape
    H = att_src.shape[0]
    bB = 512
    grid = (B // bB,)
    bias2 = bias.reshape(1, D)
    w3 = W.reshape(D, H, D)
    wa = jnp.concatenate([
        jnp.einsum('dhe,he->dh', w3, att_src),
        jnp.einsum('dhe,he->dh', w3, att_dst),
    ], axis=1)
    e_mat = jnp.repeat(jnp.eye(H, dtype=jnp.float32), D, axis=1)

    x_out, fused = pl.pallas_call(
        _gat_fused_kernel,
        grid=grid,
        in_specs=[
            pl.BlockSpec((S, bB, D), lambda i: (0, i, 0)),
            pl.BlockSpec((_SH, bB, D), lambda i: (0, i, 0)),
            pl.BlockSpec((D, H * D), lambda i: (0, 0)),
            pl.BlockSpec((D, 2 * H), lambda i: (0, 0)),
            pl.BlockSpec((H, H * D), lambda i: (0, 0)),
            pl.BlockSpec((1, D), lambda i: (0, 0)),
        ],
        out_specs=[
            pl.BlockSpec((bB, _NODES, D), lambda i: (i, 0, 0)),
            pl.BlockSpec((bB, D), lambda i: (i, 0)),
        ],
        out_shape=[
            jax.ShapeDtypeStruct((B, _NODES, D), jnp.float32),
            jax.ShapeDtypeStruct((B, D), jnp.float32),
        ],
    )(specific_features, shared_features, W, wa, e_mat, bias2)
    return fused, x_out


# speedup vs baseline: 1.1179x; 1.0770x over previous
import jax
import jax.numpy as jnp
from jax.experimental import pallas as pl

_S = 3
_SH = 3
_NODES = _S + _SH
_NBRS = (
    (1, 2, 3, 0),
    (0, 2, 4, 1),
    (1, 0, 5, 2),
    (0, 3),
    (1, 4),
    (2, 5),
)


_OUT = (
    (0, 1, 2, 3),
    (0, 1, 2, 4),
    (0, 1, 2, 5),
    (0, 3),
    (1, 4),
    (2, 5),
)


def _gat_fused_kernel(spec_ref, shr_ref, w_ref, wa_ref, e_ref, bias_ref,
                      xout_ref, fused_ref):
    bB = spec_ref.shape[1]
    D = spec_ref.shape[2]
    H = wa_ref.shape[1] // 2
    w = w_ref[...]
    wa = wa_ref[...]

    def x_node(n):
        if n < _S:
            return spec_ref[n]
        return shr_ref[n - _S]

    a_src = []
    a_dst = []
    for n in range(_NODES):
        a = jnp.dot(x_node(n), wa, preferred_element_type=jnp.float32)
        a_src.append(a[:, 0:H])
        a_dst.append(a[:, H:2 * H])

    coef = {}
    for i in range(_NODES):
        srcs = _NBRS[i]
        ad = a_dst[i]
        al = []
        for j in srcs:
            v = a_src[j] + ad
            al.append(jnp.where(v >= 0, v, 0.2 * v))
        m = al[0]
        for a in al[1:]:
            m = jnp.maximum(m, a)
        exs = [jnp.exp(a - m) for a in al]
        den = exs[0]
        for e in exs[1:]:
            den = den + e
        inv = 1.0 / (den + 1e-16)
        for j, ex in zip(srcs, exs):
            coef[(i, j)] = ex * inv

    def headsum(v):
        s = v[:, 0:D]
        for head in range(1, H):
            s = s + v[:, head * D:(head + 1) * D]
        return s

    pivot = (0, 0, 0, 0, 1, 2)
    hv = {}
    diffs = {}
    acc = [None] * _NODES
    for j in range(_NODES):
        h_j = jnp.dot(x_node(j), w, preferred_element_type=jnp.float32)
        if j < _S:
            hv[j] = h_j
        hs_j = None
        for i in _OUT[j]:
            p = pivot[i]
            if p == j:
                if hs_j is None:
                    hs_j = headsum(h_j)
                acc[i] = hs_j
                continue
            dk = (j, p)
            if dk not in diffs:
                diffs[dk] = h_j - hv[p]
            cb = jnp.dot(coef[(i, j)], e_ref[...],
                         preferred_element_type=jnp.float32)
            acc[i] = acc[i] + headsum(cb * diffs[dk])

    inv_h = 1.0 / H
    acc_mean = None
    for i in range(_NODES):
        out_i = acc[i] * inv_h + bias_ref[...]
        out_i = jnp.where(out_i > 0, out_i, jnp.exp(jnp.minimum(out_i, 0.0)) - 1.0)
        xout_ref[:, i, :] = out_i
        acc_mean = out_i if acc_mean is None else acc_mean + out_i
    fused_ref[...] = acc_mean * (1.0 / _NODES)


def kernel(specific_features, shared_features, W, att_src, att_dst, bias):
    S, B, D = specific_features.shape
    H = att_src.shape[0]
    bB = 512
    grid = (B // bB,)
    bias2 = bias.reshape(1, D)
    w3 = W.reshape(D, H, D)
    wa = jnp.concatenate([
        jnp.einsum('dhe,he->dh', w3, att_src),
        jnp.einsum('dhe,he->dh', w3, att_dst),
    ], axis=1)
    e_mat = jnp.repeat(jnp.eye(H, dtype=jnp.float32), D, axis=1)

    x_out, fused = pl.pallas_call(
        _gat_fused_kernel,
        grid=grid,
        in_specs=[
            pl.BlockSpec((S, bB, D), lambda i: (0, i, 0)),
            pl.BlockSpec((_SH, bB, D), lambda i: (0, i, 0)),
            pl.BlockSpec((D, H * D), lambda i: (0, 0)),
            pl.BlockSpec((D, 2 * H), lambda i: (0, 0)),
            pl.BlockSpec((H, H * D), lambda i: (0, 0)),
            pl.BlockSpec((1, D), lambda i: (0, 0)),
        ],
        out_specs=[
            pl.BlockSpec((bB, _NODES, D), lambda i: (i, 0, 0)),
            pl.BlockSpec((bB, D), lambda i: (i, 0)),
        ],
        out_shape=[
            jax.ShapeDtypeStruct((B, _NODES, D), jnp.float32),
            jax.ShapeDtypeStruct((B, D), jnp.float32),
        ],
    )(specific_features, shared_features, W, wa, e_mat, bias2)
    return fused, x_out


# speedup vs baseline: 1.1233x; 1.0049x over previous
import jax
import jax.numpy as jnp
from jax.experimental import pallas as pl

_S = 3
_SH = 3
_NODES = _S + _SH
_NBRS = (
    (1, 2, 3, 0),
    (0, 2, 4, 1),
    (1, 0, 5, 2),
    (0, 3),
    (1, 4),
    (2, 5),
)


_OUT = (
    (0, 1, 2, 3),
    (0, 1, 2, 4),
    (0, 1, 2, 5),
    (0, 3),
    (1, 4),
    (2, 5),
)


def _gat_fused_kernel(spec_ref, shr_ref, w_ref, wa_ref, e_ref, bias_ref,
                      xout_ref, fused_ref):
    bB = spec_ref.shape[1]
    D = spec_ref.shape[2]
    H = wa_ref.shape[1] // 2
    w = w_ref[...]
    wa = wa_ref[...]

    x_sp = spec_ref[...].reshape(_S * bB, D)
    x_sh = shr_ref[...].reshape(_SH * bB, D)

    a_sp = jnp.dot(x_sp, wa, preferred_element_type=jnp.float32)
    a_sh = jnp.dot(x_sh, wa, preferred_element_type=jnp.float32)

    def a_block(n):
        if n < _S:
            return a_sp[n * bB:(n + 1) * bB]
        return a_sh[(n - _S) * bB:(n - _S + 1) * bB]

    a_src = []
    a_dst = []
    for n in range(_NODES):
        a = a_block(n)
        a_src.append(a[:, 0:H])
        a_dst.append(a[:, H:2 * H])

    coef = {}
    for i in range(_NODES):
        srcs = _NBRS[i]
        ad = a_dst[i]
        al = []
        for j in srcs:
            v = a_src[j] + ad
            al.append(jnp.where(v >= 0, v, 0.2 * v))
        m = al[0]
        for a in al[1:]:
            m = jnp.maximum(m, a)
        exs = [jnp.exp(a - m) for a in al]
        den = exs[0]
        for e in exs[1:]:
            den = den + e
        inv = 1.0 / (den + 1e-16)
        for j, ex in zip(srcs, exs):
            coef[(i, j)] = ex * inv

    def headsum(v):
        s = v[:, 0:D]
        for head in range(1, H):
            s = s + v[:, head * D:(head + 1) * D]
        return s

    pivot = (0, 0, 0, 0, 1, 2)
    h_sp = jnp.dot(x_sp, w, preferred_element_type=jnp.float32)
    h_sh = jnp.dot(x_sh, w, preferred_element_type=jnp.float32)
    hv = {}
    diffs = {}
    acc = [None] * _NODES
    for j in range(_NODES):
        if j < _S:
            h_j = h_sp[j * bB:(j + 1) * bB]
        else:
            h_j = h_sh[(j - _S) * bB:(j - _S + 1) * bB]
        if j < _S:
            hv[j] = h_j
        hs_j = None
        for i in _OUT[j]:
            p = pivot[i]
            if p == j:
                if hs_j is None:
                    hs_j = headsum(h_j)
                acc[i] = hs_j
                continue
            dk = (j, p)
            if dk not in diffs:
                diffs[dk] = h_j - hv[p]
            cb = jnp.dot(coef[(i, j)], e_ref[...],
                         preferred_element_type=jnp.float32)
            acc[i] = acc[i] + headsum(cb * diffs[dk])

    inv_h = 1.0 / H
    acc_mean = None
    for i in range(_NODES):
        out_i = acc[i] * inv_h + bias_ref[...]
        out_i = jnp.where(out_i > 0, out_i, jnp.exp(jnp.minimum(out_i, 0.0)) - 1.0)
        xout_ref[:, i, :] = out_i
        acc_mean = out_i if acc_mean is None else acc_mean + out_i
    fused_ref[...] = acc_mean * (1.0 / _NODES)


def kernel(specific_features, shared_features, W, att_src, att_dst, bias):
    S, B, D = specific_features.shape
    H = att_src.shape[0]
    bB = 512
    grid = (B // bB,)
    bias2 = bias.reshape(1, D)
    w3 = W.reshape(D, H, D)
    wa = jnp.concatenate([
        jnp.einsum('dhe,he->dh', w3, att_src),
        jnp.einsum('dhe,he->dh', w3, att_dst),
    ], axis=1)
    e_mat = jnp.repeat(jnp.eye(H, dtype=jnp.float32), D, axis=1)

    x_out, fused = pl.pallas_call(
        _gat_fused_kernel,
        grid=grid,
        in_specs=[
            pl.BlockSpec((S, bB, D), lambda i: (0, i, 0)),
            pl.BlockSpec((_SH, bB, D), lambda i: (0, i, 0)),
            pl.BlockSpec((D, H * D), lambda i: (0, 0)),
            pl.BlockSpec((D, 2 * H), lambda i: (0, 0)),
            pl.BlockSpec((H, H * D), lambda i: (0, 0)),
            pl.BlockSpec((1, D), lambda i: (0, 0)),
        ],
        out_specs=[
            pl.BlockSpec((bB, _NODES, D), lambda i: (i, 0, 0)),
            pl.BlockSpec((bB, D), lambda i: (i, 0)),
        ],
        out_shape=[
            jax.ShapeDtypeStruct((B, _NODES, D), jnp.float32),
            jax.ShapeDtypeStruct((B, D), jnp.float32),
        ],
    )(specific_features, shared_features, W, wa, e_mat, bias2)
    return fused, x_out


# speedup vs baseline: 1.1237x; 1.0003x over previous
import jax
import jax.numpy as jnp
from jax.experimental import pallas as pl
from jax.experimental.pallas import tpu as pltpu

_S = 3
_SH = 3
_NODES = _S + _SH
_NBRS = (
    (1, 2, 3, 0),
    (0, 2, 4, 1),
    (1, 0, 5, 2),
    (0, 3),
    (1, 4),
    (2, 5),
)


_OUT = (
    (0, 1, 2, 3),
    (0, 1, 2, 4),
    (0, 1, 2, 5),
    (0, 3),
    (1, 4),
    (2, 5),
)


def _gat_fused_kernel(spec_ref, shr_ref, w_ref, wa_ref, e_ref, bias_ref,
                      xout_ref, fused_ref):
    bB = spec_ref.shape[1]
    D = spec_ref.shape[2]
    H = wa_ref.shape[1] // 2
    w = w_ref[...]
    wa = wa_ref[...]

    x_sp = spec_ref[...].reshape(_S * bB, D)
    x_sh = shr_ref[...].reshape(_SH * bB, D)

    a_sp = jnp.dot(x_sp, wa, preferred_element_type=jnp.float32)
    a_sh = jnp.dot(x_sh, wa, preferred_element_type=jnp.float32)

    def a_block(n):
        if n < _S:
            return a_sp[n * bB:(n + 1) * bB]
        return a_sh[(n - _S) * bB:(n - _S + 1) * bB]

    a_src = []
    a_dst = []
    for n in range(_NODES):
        a = a_block(n)
        a_src.append(a[:, 0:H])
        a_dst.append(a[:, H:2 * H])

    coef = {}
    for i in range(_NODES):
        srcs = _NBRS[i]
        ad = a_dst[i]
        al = []
        for j in srcs:
            v = a_src[j] + ad
            al.append(jnp.where(v >= 0, v, 0.2 * v))
        m = al[0]
        for a in al[1:]:
            m = jnp.maximum(m, a)
        exs = [jnp.exp(a - m) for a in al]
        den = exs[0]
        for e in exs[1:]:
            den = den + e
        inv = 1.0 / (den + 1e-16)
        for j, ex in zip(srcs, exs):
            coef[(i, j)] = ex * inv

    def headsum(v):
        s = v[:, 0:D]
        for head in range(1, H):
            s = s + v[:, head * D:(head + 1) * D]
        return s

    pivot = (0, 0, 0, 0, 1, 2)
    h_sp = jnp.dot(x_sp, w, preferred_element_type=jnp.float32)
    h_sh = jnp.dot(x_sh, w, preferred_element_type=jnp.float32)
    hv = {}
    diffs = {}
    acc = [None] * _NODES
    for j in range(_NODES):
        if j < _S:
            h_j = h_sp[j * bB:(j + 1) * bB]
        else:
            h_j = h_sh[(j - _S) * bB:(j - _S + 1) * bB]
        if j < _S:
            hv[j] = h_j
        hs_j = None
        for i in _OUT[j]:
            p = pivot[i]
            if p == j:
                if hs_j is None:
                    hs_j = headsum(h_j)
                acc[i] = hs_j
                continue
            dk = (j, p)
            if dk not in diffs:
                diffs[dk] = h_j - hv[p]
            cb = jnp.dot(coef[(i, j)], e_ref[...],
                         preferred_element_type=jnp.float32)
            acc[i] = acc[i] + headsum(cb * diffs[dk])

    inv_h = 1.0 / H
    acc_mean = None
    for i in range(_NODES):
        out_i = acc[i] * inv_h + bias_ref[...]
        out_i = jnp.where(out_i > 0, out_i, jnp.exp(jnp.minimum(out_i, 0.0)) - 1.0)
        xout_ref[:, i, :] = out_i
        acc_mean = out_i if acc_mean is None else acc_mean + out_i
    fused_ref[...] = acc_mean * (1.0 / _NODES)


def kernel(specific_features, shared_features, W, att_src, att_dst, bias):
    S, B, D = specific_features.shape
    H = att_src.shape[0]
    bB = 512
    grid = (B // bB,)
    bias2 = bias.reshape(1, D)
    w3 = W.reshape(D, H, D)
    wa = jnp.concatenate([
        jnp.einsum('dhe,he->dh', w3, att_src),
        jnp.einsum('dhe,he->dh', w3, att_dst),
    ], axis=1)
    e_mat = jnp.repeat(jnp.eye(H, dtype=jnp.float32), D, axis=1)

    x_out, fused = pl.pallas_call(
        _gat_fused_kernel,
        grid=grid,
        in_specs=[
            pl.BlockSpec((S, bB, D), lambda i: (0, i, 0)),
            pl.BlockSpec((_SH, bB, D), lambda i: (0, i, 0)),
            pl.BlockSpec((D, H * D), lambda i: (0, 0)),
            pl.BlockSpec((D, 2 * H), lambda i: (0, 0)),
            pl.BlockSpec((H, H * D), lambda i: (0, 0)),
            pl.BlockSpec((1, D), lambda i: (0, 0)),
        ],
        out_specs=[
            pl.BlockSpec((bB, _NODES, D), lambda i: (i, 0, 0)),
            pl.BlockSpec((bB, D), lambda i: (i, 0)),
        ],
        out_shape=[
            jax.ShapeDtypeStruct((B, _NODES, D), jnp.float32),
            jax.ShapeDtypeStruct((B, D), jnp.float32),
        ],
        compiler_params=pltpu.CompilerParams(
            dimension_semantics=("parallel",)),
    )(specific_features, shared_features, W, wa, e_mat, bias2)
    return fused, x_out
